# baseline (speedup 1.0000x reference)
import functools

import jax
import jax.numpy as jnp
from jax import lax
from jax.experimental import pallas as pl
from jax.experimental.pallas import tpu as pltpu
from jax.experimental.pallas import tpu_sc as plsc

N, D, K, O = 10000, 128, 32, 128
NC, NS, L = 2, 16, 16
NW = NC * NS
C = 4
CK = C * K
NCHUNK = N // C
NG = (NCHUNK + NW - 1) // NW
DV = D // L
NBUF = 4

_mesh = plsc.VectorSubcoreMesh(core_axis_name="c", subcore_axis_name="s")


@functools.partial(
    pl.kernel,
    out_type=jax.ShapeDtypeStruct((N, D), jnp.float32),
    mesh=_mesh,
    scratch_types=[
        pltpu.VMEM((NBUF, CK), jnp.int32),
        pltpu.VMEM((NBUF, CK, D), jnp.float32),
        pltpu.VMEM((NBUF, C, D), jnp.float32),
    ] + [pltpu.SemaphoreType.DMA] * (3 * NBUF),
)
def _gather_sum(adj_hbm, x_hbm, out_hbm, idx_v, rows_v, acc_v, *sems):
    isem = sems[0:NBUF]
    gsem = sems[NBUF:2 * NBUF]
    ssem = sems[2 * NBUF:3 * NBUF]
    w = lax.axis_index("s") * NC + lax.axis_index("c")
    ngw = (NCHUNK - w + NW - 1) // NW

    def base_of(gg):
        return (gg * NW + w) * C

    def fetch_idx(gg, b):
        return pltpu.make_async_copy(
            adj_hbm.at[pl.ds(base_of(gg) * K, CK)], idx_v.at[b], isem[b])

    def gather(gg, b):
        return pltpu.make_async_copy(
            x_hbm.at[idx_v.at[b]], rows_v.at[b], gsem[b])

    def store(gg, b):
        return pltpu.make_async_copy(
            acc_v.at[b], out_hbm.at[pl.ds(base_of(gg), C)], ssem[b])

    for b in range(NBUF):
        fetch_idx(b, b).start()
    for b in range(NBUF - 1):
        fetch_idx(b, b).wait()
        gather(b, b).start()

    def chunk_body(gg, b):
        gather(gg, b).wait()

        @pl.when(gg + NBUF < ngw)
        def _():
            fetch_idx(gg + NBUF, b).start()

        pre = gg + NBUF - 1
        bpre = (b + NBUF - 1) % NBUF

        @pl.when(pre < ngw)
        def _():
            fetch_idx(pre, bpre).wait()
            gather(pre, bpre).start()

        @pl.when(gg >= NBUF)
        def _():
            store(gg - NBUF, b).wait()

        def node_body(c, carry):
            def kstep(kk, acc):
                a = acc
                for u in range(4):
                    r = c * K + kk * 4 + u
                    a = tuple(a[d] + rows_v[b, r, pl.ds(d * L, L)]
                              for d in range(DV))
                return a
            acc0 = tuple(jnp.zeros((L,), jnp.float32) for _ in range(DV))
            accs = lax.fori_loop(0, K // 4, kstep, acc0)
            for d in range(DV):
                acc_v[b, c, pl.ds(d * L, L)] = accs[d]
            return carry

        lax.fori_loop(0, C, node_body, 0)

        store(gg, b).start()

    def quad_body(i, carry):
        for b in range(NBUF):
            gg = i * NBUF + b

            @pl.when(gg < ngw)
            def _():
                chunk_body(gg, b)
        return carry

    lax.fori_loop(0, (NG + NBUF - 1) // NBUF, quad_body, 0)

    for b in range(NBUF):
        gl = ngw - NBUF + jnp.remainder(b - (ngw - NBUF), NBUF)
        store(gl, b).wait()


RB = 10000
_DN = (((1,), (1,)), ((), ()))


def _linear_body(x_ref, agg_ref, w_ref, b_ref, o_ref):
    h = (lax.dot_general(x_ref[...], w_ref[:, :D], _DN,
                         preferred_element_type=jnp.float32)
         + lax.dot_general(agg_ref[...] * (1.0 / K), w_ref[:, D:], _DN,
                           preferred_element_type=jnp.float32)
         + b_ref[...])
    o_ref[...] = jnp.maximum(h, 0.0)


@jax.jit
def kernel(x, adj_lists, W, b):
    agg_sum = _gather_sum(adj_lists.reshape(-1), x)
    out = pl.pallas_call(
        _linear_body,
        grid=(N // RB,),
        in_specs=[
            pl.BlockSpec((RB, D), lambda i: (i, 0)),
            pl.BlockSpec((RB, D), lambda i: (i, 0)),
            pl.BlockSpec((O, 2 * D), lambda i: (0, 0)),
            pl.BlockSpec((1, O), lambda i: (0, 0)),
        ],
        out_specs=pl.BlockSpec((RB, O), lambda i: (i, 0)),
        out_shape=jax.ShapeDtypeStruct((N, O), jnp.float32),
    )(x, agg_sum, W, b.reshape(1, O))
    return out


# speedup vs baseline: 1.0052x; 1.0052x over previous
import functools

import jax
import jax.numpy as jnp
from jax import lax
from jax.experimental import pallas as pl
from jax.experimental.pallas import tpu as pltpu
from jax.experimental.pallas import tpu_sc as plsc

N, D, K, O = 10000, 128, 32, 128
NC, NS, L = 2, 16, 16
NW = NC * NS
C = 4
CK = C * K
NCHUNK = N // C
NG = (NCHUNK + NW - 1) // NW
DV = D // L
NBUF = 4

_mesh = plsc.VectorSubcoreMesh(core_axis_name="c", subcore_axis_name="s")


@functools.partial(
    pl.kernel,
    out_type=jax.ShapeDtypeStruct((N, D), jnp.float32),
    mesh=_mesh,
    scratch_types=[
        pltpu.VMEM((NBUF, CK), jnp.int32),
        pltpu.VMEM((NBUF, CK, D), jnp.float32),
        pltpu.VMEM((NBUF, C, D), jnp.float32),
    ] + [pltpu.SemaphoreType.DMA] * (3 * NBUF),
)
def _gather_sum(adj_hbm, x_hbm, out_hbm, idx_v, rows_v, acc_v, *sems):
    isem = sems[0:NBUF]
    gsem = sems[NBUF:2 * NBUF]
    ssem = sems[2 * NBUF:3 * NBUF]
    w = lax.axis_index("s") * NC + lax.axis_index("c")
    ngw = (NCHUNK - w + NW - 1) // NW

    def base_of(gg):
        return (gg * NW + w) * C

    def fetch_idx(gg, b):
        return pltpu.make_async_copy(
            adj_hbm.at[pl.ds(base_of(gg) * K, CK)], idx_v.at[b], isem[b])

    def gather(gg, b):
        return pltpu.make_async_copy(
            x_hbm.at[idx_v.at[b]], rows_v.at[b], gsem[b])

    def store(gg, b):
        return pltpu.make_async_copy(
            acc_v.at[b], out_hbm.at[pl.ds(base_of(gg), C)], ssem[b])

    for b in range(NBUF):
        fetch_idx(b, b).start()
    for b in range(NBUF - 1):
        fetch_idx(b, b).wait()
        gather(b, b).start()

    def chunk_body(gg, b):
        gather(gg, b).wait()

        @pl.when(gg + NBUF < ngw)
        def _():
            fetch_idx(gg + NBUF, b).start()

        pre = gg + NBUF - 1
        bpre = (b + NBUF - 1) % NBUF

        @pl.when(pre < ngw)
        def _():
            fetch_idx(pre, bpre).wait()
            gather(pre, bpre).start()

        @pl.when(gg >= NBUF)
        def _():
            store(gg - NBUF, b).wait()

        def node_body(c, carry):
            def kstep(kk, acc):
                a = acc
                for u in range(4):
                    r = c * K + kk * 4 + u
                    a = tuple(a[d] + rows_v[b, r, pl.ds(d * L, L)]
                              for d in range(DV))
                return a
            acc0 = tuple(jnp.zeros((L,), jnp.float32) for _ in range(DV))
            accs = lax.fori_loop(0, K // 4, kstep, acc0)
            for d in range(DV):
                acc_v[b, c, pl.ds(d * L, L)] = accs[d]
            return carry

        lax.fori_loop(0, C, node_body, 0)

        store(gg, b).start()

    def quad_body(i, carry):
        for b in range(NBUF):
            gg = i * NBUF + b

            @pl.when(gg < ngw)
            def _():
                chunk_body(gg, b)
        return carry

    lax.fori_loop(0, (NG + NBUF - 1) // NBUF, quad_body, 0)

    for b in range(NBUF):
        gl = ngw - NBUF + jnp.remainder(b - (ngw - NBUF), NBUF)
        store(gl, b).wait()


RB = 5000
_DN = (((1,), (1,)), ((), ()))


def _linear_body(x_ref, agg_ref, w_ref, b_ref, o_ref):
    h = (lax.dot_general(x_ref[...], w_ref[:, :D], _DN,
                         preferred_element_type=jnp.float32)
         + lax.dot_general(agg_ref[...] * (1.0 / K), w_ref[:, D:], _DN,
                           preferred_element_type=jnp.float32)
         + b_ref[...])
    o_ref[...] = jnp.maximum(h, 0.0)


@jax.jit
def kernel(x, adj_lists, W, b):
    agg_sum = _gather_sum(adj_lists.reshape(-1), x)
    out = pl.pallas_call(
        _linear_body,
        grid=(N // RB,),
        in_specs=[
            pl.BlockSpec((RB, D), lambda i: (i, 0)),
            pl.BlockSpec((RB, D), lambda i: (i, 0)),
            pl.BlockSpec((O, 2 * D), lambda i: (0, 0)),
            pl.BlockSpec((1, O), lambda i: (0, 0)),
        ],
        out_specs=pl.BlockSpec((RB, O), lambda i: (i, 0)),
        out_shape=jax.ShapeDtypeStruct((N, O), jnp.float32),
    )(x, agg_sum, W, b.reshape(1, O))
    return out


# speedup vs baseline: 1.0053x; 1.0001x over previous
import functools

import jax
import jax.numpy as jnp
from jax import lax
from jax.experimental import pallas as pl
from jax.experimental.pallas import tpu as pltpu
from jax.experimental.pallas import tpu_sc as plsc

N, D, K, O = 10000, 128, 32, 128
NC, NS, L = 2, 16, 16
NW = NC * NS
C = 8
CH = C // 2
CK = CH * K
NCHUNK = N // C
NG = (NCHUNK + NW - 1) // NW
DV = D // L
NBUF = 3

_mesh = plsc.VectorSubcoreMesh(core_axis_name="c", subcore_axis_name="s")


@functools.partial(
    pl.kernel,
    out_type=jax.ShapeDtypeStruct((N, D), jnp.float32),
    mesh=_mesh,
    scratch_types=[
        pltpu.VMEM((2 * NBUF, CK), jnp.int32),
        pltpu.VMEM((2 * NBUF, CK, D), jnp.float32),
        pltpu.VMEM((NBUF, C, D), jnp.float32),
    ] + [pltpu.SemaphoreType.DMA] * (4 * NBUF),
)
def _gather_sum(adj_hbm, x_hbm, out_hbm, idx_v, rows_v, acc_v, *sems):
    isem = sems[0:NBUF]
    gsem = sems[NBUF:3 * NBUF]
    ssem = sems[3 * NBUF:4 * NBUF]
    w = lax.axis_index("s") * NC + lax.axis_index("c")
    ngw = (NCHUNK - w + NW - 1) // NW

    def base_of(gg):
        return (gg * NW + w) * C

    def fetch_idx(gg, b):
        return pltpu.make_async_copy(
            adj_hbm.at[pl.ds((gg * NW + w) * 2, 2)],
            idx_v.at[pl.ds(2 * b, 2)], isem[b])

    def gather(gg, b, h):
        return pltpu.make_async_copy(
            x_hbm.at[idx_v.at[2 * b + h]], rows_v.at[2 * b + h],
            gsem[2 * b + h])

    def store(gg, b):
        return pltpu.make_async_copy(
            acc_v.at[b], out_hbm.at[pl.ds(base_of(gg), C)], ssem[b])

    for b in range(NBUF):
        fetch_idx(b, b).start()
    for b in range(NBUF - 1):
        fetch_idx(b, b).wait()
        gather(b, b, 0).start()
        gather(b, b, 1).start()

    def chunk_body(gg, b):
        gather(gg, b, 0).wait()
        gather(gg, b, 1).wait()

        @pl.when(gg + NBUF < ngw)
        def _():
            fetch_idx(gg + NBUF, b).start()

        pre = gg + NBUF - 1
        bpre = (b + NBUF - 1) % NBUF

        @pl.when(pre < ngw)
        def _():
            fetch_idx(pre, bpre).wait()
            gather(pre, bpre, 0).start()
            gather(pre, bpre, 1).start()

        @pl.when(gg >= NBUF)
        def _():
            store(gg - NBUF, b).wait()

        def node_body(c, carry):
            h = c // CH
            cc = c - h * CH

            def kstep(kk, acc):
                a = acc
                for u in range(4):
                    r = cc * K + kk * 4 + u
                    a = tuple(a[d] + rows_v[2 * b + h, r, pl.ds(d * L, L)]
                              for d in range(DV))
                return a
            acc0 = tuple(jnp.zeros((L,), jnp.float32) for _ in range(DV))
            accs = lax.fori_loop(0, K // 4, kstep, acc0)
            for d in range(DV):
                acc_v[b, c, pl.ds(d * L, L)] = accs[d]
            return carry

        lax.fori_loop(0, C, node_body, 0)

        store(gg, b).start()

    def ring_body(i, carry):
        for b in range(NBUF):
            gg = i * NBUF + b

            @pl.when(gg < ngw)
            def _():
                chunk_body(gg, b)
        return carry

    lax.fori_loop(0, (NG + NBUF - 1) // NBUF, ring_body, 0)

    for b in range(NBUF):
        gl = ngw - NBUF + jnp.remainder(b - (ngw - NBUF), NBUF)
        store(gl, b).wait()


RB = 5000
_DN = (((1,), (1,)), ((), ()))


def _linear_body(x_ref, agg_ref, w_ref, b_ref, o_ref):
    h = (lax.dot_general(x_ref[...], w_ref[:, :D], _DN,
                         preferred_element_type=jnp.float32)
         + lax.dot_general(agg_ref[...] * (1.0 / K), w_ref[:, D:], _DN,
                           preferred_element_type=jnp.float32)
         + b_ref[...])
    o_ref[...] = jnp.maximum(h, 0.0)


@jax.jit
def kernel(x, adj_lists, W, b):
    agg_sum = _gather_sum(adj_lists.reshape(N * K // CK, CK), x)
    out = pl.pallas_call(
        _linear_body,
        grid=(N // RB,),
        in_specs=[
            pl.BlockSpec((RB, D), lambda i: (i, 0)),
            pl.BlockSpec((RB, D), lambda i: (i, 0)),
            pl.BlockSpec((O, 2 * D), lambda i: (0, 0)),
            pl.BlockSpec((1, O), lambda i: (0, 0)),
        ],
        out_specs=pl.BlockSpec((RB, O), lambda i: (i, 0)),
        out_shape=jax.ShapeDtypeStruct((N, O), jnp.float32),
    )(x, agg_sum, W, b.reshape(1, O))
    return out
